# baseline (speedup 1.0000x reference)
import functools

import jax
import jax.numpy as jnp
from jax import lax
from jax.experimental import pallas as pl
from jax.experimental.pallas import tpu as pltpu
from jax.experimental.pallas import tpu_sc as plsc

_MIN_NORM = 1e-15
_PROJ_EPS = 4e-3

_N = 8192
_D = 128
_DOUT = 16
_NUM_ADJS = 2

_BM = 256

_R_SC = 1024
_NSUB = 32
_RPS = _R_SC // _NSUB
_KC = 2048
_NKC = _N // _KC


def _support_body(x_ref, w_ref, s_ref, st_ref):
    x = x_ref[...]
    n = jnp.maximum(jnp.sqrt(jnp.sum(x * x, axis=-1, keepdims=True)), _MIN_NORM)
    t = jnp.clip(n, -1.0 + 1e-7, 1.0 - 1e-7)
    atanh_t = 0.5 * (jnp.log1p(t) - jnp.log1p(-t))
    xt = (atanh_t / n) * x
    w = w_ref[...]
    s_ref[...] = jnp.dot(xt, w, preferred_element_type=jnp.float32)
    for a in range(_NUM_ADJS):
        wa = w[:, a * _DOUT:(a + 1) * _DOUT]
        st_ref[a] = lax.dot_general(wa, xt, (((0,), (1,)), ((), ())),
                                    preferred_element_type=jnp.float32)


def _expmap_proj(u):
    n = jnp.maximum(jnp.sqrt(jnp.sum(u * u, axis=-1, keepdims=True)), _MIN_NORM)
    e = jnp.tanh(n) * u / n
    rn = jnp.maximum(jnp.sqrt(jnp.sum(e * e, axis=-1, keepdims=True)), _MIN_NORM)
    maxnorm = 1.0 - _PROJ_EPS
    return jnp.where(rn > maxnorm, e / rn * maxnorm, e)


def _main_body(s_ref, b_ref, adj_ref, out_ref):
    a = adj_ref[...]
    s = s_ref[...]
    u0 = jnp.dot(a[0], s[:, :_DOUT], preferred_element_type=jnp.float32)
    u1 = jnp.dot(a[1], s[:, _DOUT:], preferred_element_type=jnp.float32)
    out_ref[...] = (_expmap_proj(u0) + _expmap_proj(u1)) / 11.0 + b_ref[...]


def _sc_tail_kernel(adj_hbm, st_hbm, part_hbm,
                    st_v, arow_v, part_v, sem_st, sem_row):
    wid = lax.axis_index("c") * 16 + lax.axis_index("s")
    base_row = (_N - _R_SC) + wid * _RPS
    npairs = _RPS // 2

    def zbody(i, _):
        part_v[pl.ds(i * 16, 16)] = jnp.zeros((16,), jnp.float32)
        return 0
    lax.fori_loop(0, _NUM_ADJS * _RPS * _DOUT, zbody, 0, unroll=False)

    def row_dma(pair_idx, slot, k0):
        g = base_row + pair_idx * 2
        return [pltpu.make_async_copy(
                    adj_hbm.at[a, pl.ds(g, 2), pl.ds(k0, _KC)],
                    arow_v.at[slot, a], sem_row)
                for a in range(_NUM_ADJS)]

    def compute_pair(slot, r0, kc_is_first_ignored):
        for rr in range(2):
            for a in range(_NUM_ADJS):
                zero = jnp.zeros((16,), jnp.float32)
                init = tuple(zero for _ in range(_DOUT))

                def jbody(j, acc, a=a, rr=rr, slot=slot):
                    jk = j * 16
                    av = arow_v[slot, a, rr, pl.ds(jk, 16)]
                    return tuple(acc[c] + av * st_v[a, c, pl.ds(jk, 16)]
                                 for c in range(_DOUT))

                accs = lax.fori_loop(0, _KC // 16, jbody, init, unroll=4)
                for c in range(_DOUT):
                    off = (((a * _RPS) + r0 + rr) * _DOUT + c) * 16
                    plsc.addupdate(part_v.at[pl.ds(off, 16)], accs[c])

    last = npairs - 1

    def kc_body(kc, _):
        k0 = kc * _KC
        pltpu.async_copy(st_hbm.at[:, :, pl.ds(k0, _KC)], st_v, sem_st).wait()

        for cp in row_dma(0, 0, k0):
            cp.start()

        def qbody(q, _, k0=k0):
            pa = q * 2
            pb = jnp.minimum(pa + 1, last)
            pc = jnp.minimum(pa + 2, last)
            for cp in row_dma(pa, 0, k0):
                cp.wait()
            for cp in row_dma(pb, 1, k0):
                cp.start()
            compute_pair(0, pa * 2, None)
            for cp in row_dma(pb, 1, k0):
                cp.wait()
            for cp in row_dma(pc, 0, k0):
                cp.start()
            compute_pair(1, pb * 2, None)
            return 0

        lax.fori_loop(0, npairs // 2, qbody, 0, unroll=False)
        for cp in row_dma(last, 0, k0):
            cp.wait()
        return 0

    lax.fori_loop(0, _NKC, kc_body, 0, unroll=False)

    blk = _RPS * _DOUT * 16
    for a in range(_NUM_ADJS):
        pltpu.sync_copy(
            part_v.at[pl.ds(a * blk, blk)],
            part_hbm.at[pl.ds((a * _R_SC + wid * _RPS) * _DOUT * 16, blk)])


def _sc_tail_partials(adj, st):
    mesh = plsc.VectorSubcoreMesh(core_axis_name="c", subcore_axis_name="s")
    fn = functools.partial(
        pl.kernel, mesh=mesh,
        out_type=jax.ShapeDtypeStruct((_NUM_ADJS * _R_SC * _DOUT * 16,),
                                      jnp.float32),
        scratch_types=[
            pltpu.VMEM((_NUM_ADJS, _DOUT, _KC), jnp.float32),
            pltpu.VMEM((2, _NUM_ADJS, 2, _KC), jnp.float32),
            pltpu.VMEM((_NUM_ADJS * _RPS * _DOUT * 16,), jnp.float32),
            pltpu.SemaphoreType.DMA,
            pltpu.SemaphoreType.DMA,
        ],
    )(_sc_tail_kernel)
    return fn(adj, st)


def _tail_epilogue_body(p_ref, b_ref, out_ref):
    p = p_ref[...]
    rows = lax.broadcasted_iota(jnp.int32, (_DOUT * 16, _DOUT), 0)
    cols = lax.broadcasted_iota(jnp.int32, (_DOUT * 16, _DOUT), 1)
    m = jnp.where(rows // 16 == cols, 1.0, 0.0).astype(jnp.float32)
    u = jnp.dot(p, m, preferred_element_type=jnp.float32)
    e = _expmap_proj(u)
    out_ref[...] = (e[:_R_SC] + e[_R_SC:]) / 11.0 + b_ref[...]


def kernel(x, adj, adj_weight, bias):
    w2 = jnp.transpose(adj_weight, (1, 0, 2)).reshape(_D, _NUM_ADJS * _DOUT)

    s, st = pl.pallas_call(
        _support_body,
        out_shape=(
            jax.ShapeDtypeStruct((_N, _NUM_ADJS * _DOUT), jnp.float32),
            jax.ShapeDtypeStruct((_NUM_ADJS, _DOUT, _N), jnp.float32),
        ),
    )(x, w2)

    part = _sc_tail_partials(adj, st).reshape(_NUM_ADJS * _R_SC, _DOUT * 16)
    out_tail = pl.pallas_call(
        _tail_epilogue_body,
        out_shape=jax.ShapeDtypeStruct((_R_SC, _DOUT), jnp.float32),
    )(part, bias.reshape(1, _DOUT))

    nr = (_N - _R_SC) // _BM
    out_head = pl.pallas_call(
        _main_body,
        grid=(nr,),
        in_specs=[
            pl.BlockSpec((_N, _NUM_ADJS * _DOUT), lambda i: (0, 0)),
            pl.BlockSpec((1, _DOUT), lambda i: (0, 0)),
            pl.BlockSpec((_NUM_ADJS, _BM, _N), lambda i: (0, i, 0)),
        ],
        out_specs=pl.BlockSpec((_BM, _DOUT), lambda i: (i, 0)),
        out_shape=jax.ShapeDtypeStruct((_N - _R_SC, _DOUT), jnp.float32),
        compiler_params=pltpu.CompilerParams(
            dimension_semantics=("arbitrary",),
        ),
    )(s, bias.reshape(1, _DOUT), adj)

    return jnp.concatenate([out_head, out_tail], axis=0)


# speedup vs baseline: 2.3094x; 2.3094x over previous
import jax
import jax.numpy as jnp
from jax.experimental import pallas as pl
from jax.experimental.pallas import tpu as pltpu

_MIN_NORM = 1e-15
_PROJ_EPS = 4e-3

_N = 8192
_D = 128
_DOUT = 16
_NUM_ADJS = 2

_BM = 256


def _expmap_proj(u):
    n = jnp.maximum(jnp.sqrt(jnp.sum(u * u, axis=-1, keepdims=True)), _MIN_NORM)
    e = jnp.tanh(n) * u / n
    rn = jnp.maximum(jnp.sqrt(jnp.sum(e * e, axis=-1, keepdims=True)), _MIN_NORM)
    maxnorm = 1.0 - _PROJ_EPS
    return jnp.where(rn > maxnorm, e / rn * maxnorm, e)


def _body(x_ref, w_ref, b_ref, adj_ref, out_ref, s_ref):
    i = pl.program_id(0)

    @pl.when(i == 0)
    def _support():
        x = x_ref[...]
        n = jnp.maximum(jnp.sqrt(jnp.sum(x * x, axis=-1, keepdims=True)),
                        _MIN_NORM)
        t = jnp.clip(n, -1.0 + 1e-7, 1.0 - 1e-7)
        atanh_t = 0.5 * (jnp.log1p(t) - jnp.log1p(-t))
        xt = (atanh_t / n) * x
        s_ref[...] = jnp.dot(xt, w_ref[...],
                             preferred_element_type=jnp.float32)

    a = adj_ref[...]
    s = s_ref[...]
    u0 = jnp.dot(a[0], s[:, :_DOUT], preferred_element_type=jnp.float32)
    u1 = jnp.dot(a[1], s[:, _DOUT:], preferred_element_type=jnp.float32)
    out_ref[...] = (_expmap_proj(u0) + _expmap_proj(u1)) / 11.0 + b_ref[...]


def kernel(x, adj, adj_weight, bias):
    w2 = jnp.transpose(adj_weight, (1, 0, 2)).reshape(_D, _NUM_ADJS * _DOUT)

    nr = _N // _BM
    out = pl.pallas_call(
        _body,
        grid=(nr,),
        in_specs=[
            pl.BlockSpec((_N, _D), lambda i: (0, 0)),
            pl.BlockSpec((_D, _NUM_ADJS * _DOUT), lambda i: (0, 0)),
            pl.BlockSpec((1, _DOUT), lambda i: (0, 0)),
            pl.BlockSpec((_NUM_ADJS, _BM, _N), lambda i: (0, i, 0)),
        ],
        out_specs=pl.BlockSpec((_BM, _DOUT), lambda i: (i, 0)),
        out_shape=jax.ShapeDtypeStruct((_N, _DOUT), jnp.float32),
        scratch_shapes=[pltpu.VMEM((_N, _NUM_ADJS * _DOUT), jnp.float32)],
        compiler_params=pltpu.CompilerParams(
            dimension_semantics=("arbitrary",),
        ),
    )(x, w2, bias.reshape(1, _DOUT), adj)
    return out
